# baseline (speedup 1.0000x reference)
import functools

import jax
import jax.numpy as jnp
from jax import lax
from jax.experimental import pallas as pl
from jax.experimental.pallas import tpu as pltpu
from jax.experimental.pallas import tpu_sc as plsc

D = 3072
K = 8192
B = 1024
KB = 512
NK = K // KB

_NC = 2
_NS = 16
_NW = _NC * _NS
_BPW = B // _NW


def _sim_argmax_body(t_ref, w_ref, idx_ref, wt_ref, tb_ref, vals_ref, idxs_ref):
    j = pl.program_id(0)

    @pl.when(j == 0)
    def _cast_t():
        tb_ref[...] = t_ref[...].astype(jnp.bfloat16)

    wb = w_ref[...].astype(jnp.bfloat16)
    a = jnp.dot(tb_ref[...], wb, preferred_element_type=jnp.float32)
    wt_ref[...] = pltpu.bitcast(wb, jnp.float32).T
    m = jnp.max(a, axis=1, keepdims=True)
    lane = lax.broadcasted_iota(jnp.int32, a.shape, 1)
    loc = jnp.min(jnp.where(a == m, lane, K), axis=1, keepdims=True) + j * KB

    @pl.when(j == 0)
    def _init():
        vals_ref[...] = m
        idxs_ref[...] = loc

    @pl.when(j > 0)
    def _update():
        better = m > vals_ref[...]
        vals_ref[...] = jnp.where(better, m, vals_ref[...])
        idxs_ref[...] = jnp.where(better, loc, idxs_ref[...])

    @pl.when(j == NK - 1)
    def _emit():
        idx_ref[...] = idxs_ref[...]


def _sim_argmax(t_bf16, W):
    return pl.pallas_call(
        _sim_argmax_body,
        grid=(NK,),
        in_specs=[
            pl.BlockSpec((B, D), lambda j: (0, 0)),
            pl.BlockSpec((D, KB), lambda j: (0, j)),
        ],
        out_specs=[
            pl.BlockSpec((B, 1), lambda j: (0, 0)),
            pl.BlockSpec((KB, D // 2), lambda j: (j, 0)),
        ],
        out_shape=[
            jax.ShapeDtypeStruct((B, 1), jnp.int32),
            jax.ShapeDtypeStruct((K, D // 2), jnp.float32),
        ],
        scratch_shapes=[
            pltpu.VMEM((B, D), jnp.bfloat16),
            pltpu.VMEM((B, 1), jnp.float32),
            pltpu.VMEM((B, 1), jnp.int32),
        ],
        compiler_params=pltpu.CompilerParams(
            dimension_semantics=("arbitrary",),
        ),
    )(t_bf16, W)


def _decode_gather(wt, idx):
    mesh = plsc.VectorSubcoreMesh(core_axis_name="c", subcore_axis_name="s")

    @functools.partial(
        pl.kernel,
        mesh=mesh,
        out_type=jax.ShapeDtypeStruct((B, D // 2), jnp.float32),
        scratch_types=[
            pltpu.VMEM((_BPW,), jnp.int32),
            pltpu.VMEM((_BPW, D // 2), jnp.float32),
            pltpu.SemaphoreType.DMA,
        ],
    )
    def gk(wt_hbm, idx_hbm, out_hbm, idx_v, rows_v, sem):
        wid = lax.axis_index("s") * _NC + lax.axis_index("c")
        base = wid * _BPW
        pltpu.sync_copy(idx_hbm.at[pl.ds(base, _BPW)], idx_v)
        pltpu.async_copy(wt_hbm.at[idx_v], rows_v, sem).wait()
        pltpu.sync_copy(rows_v, out_hbm.at[pl.ds(base, _BPW)])

    return gk(wt, idx)


def kernel(targ, W):
    t = targ.reshape(targ.shape[0], -1)
    idx2d, wt = _sim_argmax(t, W)
    packed = _decode_gather(wt, idx2d.reshape(B))
    pairs = jax.lax.bitcast_convert_type(packed, jnp.bfloat16)
    return pairs.reshape(B, D).astype(jnp.float32)


# speedup vs baseline: 1.3041x; 1.3041x over previous
import functools

import jax
import jax.numpy as jnp
from jax import lax
from jax.experimental import pallas as pl
from jax.experimental.pallas import tpu as pltpu
from jax.experimental.pallas import tpu_sc as plsc

D = 3072
K = 8192
B = 1024
KB = 512
NK = K // KB

_NC = 2
_NS = 16
_NW = _NC * _NS
_BPW = B // _NW


def _sim_argmax_body(t_ref, w_ref, idx_ref, wt_ref, tb_ref, vals_ref, idxs_ref):
    j = pl.program_id(0)

    @pl.when(j == 0)
    def _cast_t():
        tb_ref[...] = t_ref[...].astype(jnp.bfloat16)

    wb = w_ref[...].astype(jnp.bfloat16)
    a = jnp.dot(tb_ref[...], wb, preferred_element_type=jnp.float32)
    top = lax.bitcast_convert_type(wb[: D // 2, :].astype(jnp.float32),
                                   jnp.int32)
    bot = lax.bitcast_convert_type(wb[D // 2:, :].astype(jnp.float32),
                                   jnp.int32)
    packed = lax.bitwise_or(lax.shift_right_logical(top, 16), bot)
    wt_ref[...] = packed.T
    m = jnp.max(a, axis=1, keepdims=True)
    lane = lax.broadcasted_iota(jnp.int32, a.shape, 1)
    loc = jnp.min(jnp.where(a == m, lane, K), axis=1, keepdims=True) + j * KB

    @pl.when(j == 0)
    def _init():
        vals_ref[...] = m
        idxs_ref[...] = loc

    @pl.when(j > 0)
    def _update():
        better = m > vals_ref[...]
        vals_ref[...] = jnp.where(better, m, vals_ref[...])
        idxs_ref[...] = jnp.where(better, loc, idxs_ref[...])

    @pl.when(j == NK - 1)
    def _emit():
        idx_ref[...] = idxs_ref[...]


def _sim_argmax(t_bf16, W):
    return pl.pallas_call(
        _sim_argmax_body,
        grid=(NK,),
        in_specs=[
            pl.BlockSpec((B, D), lambda j: (0, 0)),
            pl.BlockSpec((D, KB), lambda j: (0, j)),
        ],
        out_specs=[
            pl.BlockSpec((B, 1), lambda j: (0, 0)),
            pl.BlockSpec((KB, D // 2), lambda j: (j, 0)),
        ],
        out_shape=[
            jax.ShapeDtypeStruct((B, 1), jnp.int32),
            jax.ShapeDtypeStruct((K, D // 2), jnp.int32),
        ],
        scratch_shapes=[
            pltpu.VMEM((B, D), jnp.bfloat16),
            pltpu.VMEM((B, 1), jnp.float32),
            pltpu.VMEM((B, 1), jnp.int32),
        ],
        compiler_params=pltpu.CompilerParams(
            dimension_semantics=("arbitrary",),
        ),
    )(t_bf16, W)


_CH = 16


def _decode_gather(wt, idx):
    mesh = plsc.VectorSubcoreMesh(core_axis_name="c", subcore_axis_name="s")

    @functools.partial(
        pl.kernel,
        mesh=mesh,
        out_type=jax.ShapeDtypeStruct((B, D), jnp.int32),
        scratch_types=[
            pltpu.VMEM((_BPW,), jnp.int32),
            pltpu.VMEM((_BPW, D // 2), jnp.int32),
            pltpu.VMEM((_CH, D), jnp.int32),
            pltpu.SemaphoreType.DMA,
        ],
    )
    def gk(wt_hbm, idx_hbm, out_hbm, idx_v, rows_v, un_v, sem):
        wid = lax.axis_index("s") * _NC + lax.axis_index("c")
        base = wid * _BPW
        pltpu.sync_copy(idx_hbm.at[pl.ds(base, _BPW)], idx_v)
        pltpu.async_copy(wt_hbm.at[idx_v], rows_v, sem).wait()

        for c in range(_BPW // _CH):
            def unpack_row(r, carry):
                for v in range(D // 32):
                    w = rows_v[c * _CH + r, pl.ds(v * 16, 16)]
                    un_v[r, pl.ds(v * 16, 16)] = w * jnp.int32(65536)
                    un_v[r, pl.ds(v * 16 + D // 2, 16)] = lax.bitwise_and(
                        w, jnp.int32(-65536))
                return carry

            lax.fori_loop(0, _CH, unpack_row, 0)
            pltpu.sync_copy(un_v, out_hbm.at[pl.ds(base + c * _CH, _CH)])

    return gk(wt, idx)


def kernel(targ, W):
    t = targ.reshape(targ.shape[0], -1)
    idx2d, wt = _sim_argmax(t, W)
    out_bits = _decode_gather(wt, idx2d.reshape(B))
    return jax.lax.bitcast_convert_type(out_bits, jnp.float32)


# speedup vs baseline: 1.4851x; 1.1387x over previous
import functools

import jax
import jax.numpy as jnp
from jax import lax
from jax.experimental import pallas as pl
from jax.experimental.pallas import tpu as pltpu
from jax.experimental.pallas import tpu_sc as plsc

D = 3072
K = 8192
B = 1024
KB = 512
NK = K // KB

_NC = 2
_NS = 16
_NW = _NC * _NS
_BPW = B // _NW


def _sim_argmax_body(t_ref, w_ref, idx_ref, wt_ref, tb_ref, vals_ref, idxs_ref):
    j = pl.program_id(0)

    @pl.when(j == 0)
    def _cast_t():
        tb_ref[...] = t_ref[...].astype(jnp.bfloat16)

    wb = w_ref[...].astype(jnp.bfloat16)
    a = jnp.dot(tb_ref[...], wb, preferred_element_type=jnp.float32)
    wt_ref[...] = w_ref[...].T
    m = jnp.max(a, axis=1, keepdims=True)
    lane = lax.broadcasted_iota(jnp.int32, a.shape, 1)
    loc = jnp.min(jnp.where(a == m, lane, K), axis=1, keepdims=True) + j * KB

    @pl.when(j == 0)
    def _init():
        vals_ref[...] = m
        idxs_ref[...] = loc

    @pl.when(j > 0)
    def _update():
        better = m > vals_ref[...]
        vals_ref[...] = jnp.where(better, m, vals_ref[...])
        idxs_ref[...] = jnp.where(better, loc, idxs_ref[...])

    @pl.when(j == NK - 1)
    def _emit():
        idx_ref[...] = idxs_ref[...]


def _sim_argmax(t_bf16, W):
    return pl.pallas_call(
        _sim_argmax_body,
        grid=(NK,),
        in_specs=[
            pl.BlockSpec((B, D), lambda j: (0, 0)),
            pl.BlockSpec((D, KB), lambda j: (0, j)),
        ],
        out_specs=[
            pl.BlockSpec((B, 1), lambda j: (0, 0)),
            pl.BlockSpec((KB, D), lambda j: (j, 0)),
        ],
        out_shape=[
            jax.ShapeDtypeStruct((B, 1), jnp.int32),
            jax.ShapeDtypeStruct((K, D), jnp.float32),
        ],
        scratch_shapes=[
            pltpu.VMEM((B, D), jnp.bfloat16),
            pltpu.VMEM((B, 1), jnp.float32),
            pltpu.VMEM((B, 1), jnp.int32),
        ],
        compiler_params=pltpu.CompilerParams(
            dimension_semantics=("arbitrary",),
        ),
    )(t_bf16, W)


_CH = 8


def _decode_gather(wt, idx):
    mesh = plsc.VectorSubcoreMesh(core_axis_name="c", subcore_axis_name="s")

    @functools.partial(
        pl.kernel,
        mesh=mesh,
        out_type=jax.ShapeDtypeStruct((B, D), jnp.float32),
        scratch_types=[
            pltpu.VMEM((_BPW,), jnp.int32),
            pltpu.VMEM((_BPW, D), jnp.float32),
            pltpu.SemaphoreType.DMA,
        ],
    )
    def gk(wt_hbm, idx_hbm, out_hbm, idx_v, rows_v, sem):
        wid = lax.axis_index("s") * _NC + lax.axis_index("c")
        base = wid * _BPW
        pltpu.sync_copy(idx_hbm.at[pl.ds(base, _BPW)], idx_v)
        pltpu.async_copy(wt_hbm.at[idx_v], rows_v, sem).wait()
        pltpu.sync_copy(rows_v, out_hbm.at[pl.ds(base, _BPW)])

    return gk(wt, idx)


def kernel(targ, W):
    t = targ.reshape(targ.shape[0], -1)
    idx2d, wt = _sim_argmax(t, W)
    return _decode_gather(wt, idx2d.reshape(B))
